# baseline (speedup 1.0000x reference)
import functools

import jax
import jax.numpy as jnp
from jax.experimental import pallas as pl
from jax.experimental.pallas import tpu as pltpu

_VMEM_LIMIT = 100 * 1024 * 1024


def _lane_shift(a, s, zeros):
    if s == 0:
        return a
    if s > 0:
        return jnp.concatenate([a[..., s:], zeros[..., :s]], axis=-1)
    return jnp.concatenate([zeros[..., :(-s)], a[..., :s]], axis=-1)


def _conv_stats_kernel(x_ref, w_ref, yt_ref, psum_ref, psq_ref, *, kk, ho, wo):
    xs = x_ref[...].astype(jnp.bfloat16)
    nb, cin, hw = xs.shape
    pad = (kk - 1) // 2
    zeros = jnp.zeros_like(xs)
    col = jax.lax.broadcasted_iota(jnp.int32, (1, 1, hw), 2) % wo
    taps = []
    for kh in range(kk):
        dh = kh - pad
        for kw in range(kk):
            dw = kw - pad
            t = _lane_shift(xs, dh * wo + dw, zeros)
            if dw < 0:
                t = jnp.where(col >= -dw, t, 0).astype(jnp.bfloat16)
            elif dw > 0:
                t = jnp.where(col < wo - dw, t, 0).astype(jnp.bfloat16)
            taps.append(t)
    pt = jnp.concatenate(taps, axis=1)
    wt = w_ref[...]
    ps, pq = 0.0, 0.0
    for j in range(nb):
        yf = jnp.dot(wt, pt[j], preferred_element_type=jnp.float32)
        yt_ref[j] = yf.astype(yt_ref.dtype)
        ps = ps + jnp.sum(yf, axis=1, keepdims=True)
        pq = pq + jnp.sum(yf * yf, axis=1, keepdims=True)
    psum_ref[...] = ps[None]
    psq_ref[...] = pq[None]


def _bn_relu_kernel(y_ref, scale_ref, shift_ref, o_ref):
    o_ref[...] = jnp.maximum(
        y_ref[...].astype(jnp.float32) * scale_ref[...][None] + shift_ref[...][None],
        0.0,
    )


@functools.partial(jax.jit, static_argnames=())
def kernel(x, w, b, gamma, beta):
    eps = 1e-5
    N, Cin, H, W = x.shape
    Cout = w.shape[0]
    K = w.shape[2]
    Ho, Wo = H, W
    HW = Ho * Wo
    M = N * HW
    KKC = K * K * Cin
    del b

    x3 = x.reshape(N, Cin, HW)
    w2dt = jnp.transpose(w, (0, 2, 3, 1)).reshape(Cout, KKC).astype(jnp.bfloat16)

    nb = 2 if N % 2 == 0 else 1
    G = N // nb
    body = functools.partial(_conv_stats_kernel, kk=K, ho=Ho, wo=Wo)
    yt, psum, psq = pl.pallas_call(
        body,
        out_shape=(
            jax.ShapeDtypeStruct((N, Cout, HW), jnp.bfloat16),
            jax.ShapeDtypeStruct((G, Cout, 1), jnp.float32),
            jax.ShapeDtypeStruct((G, Cout, 1), jnp.float32),
        ),
        grid=(G,),
        in_specs=[
            pl.BlockSpec((nb, Cin, HW), lambda i: (i, 0, 0)),
            pl.BlockSpec((Cout, KKC), lambda i: (0, 0)),
        ],
        out_specs=[
            pl.BlockSpec((nb, Cout, HW), lambda i: (i, 0, 0)),
            pl.BlockSpec((1, Cout, 1), lambda i: (i, 0, 0)),
            pl.BlockSpec((1, Cout, 1), lambda i: (i, 0, 0)),
        ],
        compiler_params=pltpu.CompilerParams(
            dimension_semantics=("parallel",),
            vmem_limit_bytes=_VMEM_LIMIT,
        ),
        cost_estimate=pl.CostEstimate(
            flops=2 * M * KKC * Cout,
            transcendentals=0,
            bytes_accessed=4 * M * Cin + 2 * KKC * Cout + 2 * M * Cout,
        ),
    )(x3, w2dt)

    inv_m = 1.0 / float(M)
    mean = jnp.sum(psum, axis=0) * inv_m
    var = jnp.maximum(jnp.sum(psq, axis=0) * inv_m - mean * mean, 0.0)
    g2d = gamma.reshape(Cout, 1).astype(jnp.float32)
    b2d = beta.reshape(Cout, 1).astype(jnp.float32)
    scale = g2d * jax.lax.rsqrt(var + eps)
    shift = b2d - mean * scale
    scale_b = jnp.broadcast_to(scale, (Cout, HW))
    shift_b = jnp.broadcast_to(shift, (Cout, HW))

    nb2 = 8
    while N % nb2:
        nb2 //= 2
    out3 = pl.pallas_call(
        _bn_relu_kernel,
        out_shape=jax.ShapeDtypeStruct((N, Cout, HW), jnp.float32),
        grid=(N // nb2,),
        in_specs=[
            pl.BlockSpec((nb2, Cout, HW), lambda i: (i, 0, 0)),
            pl.BlockSpec((Cout, HW), lambda i: (0, 0)),
            pl.BlockSpec((Cout, HW), lambda i: (0, 0)),
        ],
        out_specs=pl.BlockSpec((nb2, Cout, HW), lambda i: (i, 0, 0)),
        compiler_params=pltpu.CompilerParams(
            dimension_semantics=("parallel",),
            vmem_limit_bytes=_VMEM_LIMIT,
        ),
        cost_estimate=pl.CostEstimate(
            flops=3 * M * Cout,
            transcendentals=0,
            bytes_accessed=6 * M * Cout,
        ),
    )(yt, scale_b, shift_b)

    return out3.reshape(N, Cout, Ho, Wo)
